# baseline (speedup 1.0000x reference)
import functools

import jax
import jax.numpy as jnp
from jax import lax
from jax.experimental import pallas as pl
from jax.experimental.pallas import tpu as pltpu
from jax.experimental.pallas import tpu_sc as plsc

_REINIT_P = 0.05
_N, _R, _C = 10000, 250, 100
_B = 128
_Q = 8
_NW = 32
_J = _N // _NW
_TAIL = _N - _J * _NW


def _sc_copy_body(buf, dst, s0, s1, din0, din1, dout0, dout1):
    wid = lax.axis_index("s") * 2 + lax.axis_index("c")

    def pair(p, carry):
        r0 = (2 * p) * _NW + wid
        r1 = (2 * p + 1) * _NW + wid
        c0 = pltpu.async_copy(buf.at[pl.ds(r0, 1)], s0, din0)
        c1 = pltpu.async_copy(buf.at[pl.ds(r1, 1)], s1, din1)
        c0.wait()
        o0 = pltpu.async_copy(s0, dst.at[pl.ds(r0, 1)], dout0)
        c1.wait()
        o1 = pltpu.async_copy(s1, dst.at[pl.ds(r1, 1)], dout1)
        o0.wait()
        o1.wait()
        return carry

    jax.lax.fori_loop(0, _J // 2, pair, 0)

    @pl.when(wid < _TAIL)
    def _():
        r = _J * _NW + wid
        pltpu.async_copy(buf.at[pl.ds(r, 1)], s0, din0).wait()
        pltpu.async_copy(s0, dst.at[pl.ds(r, 1)], dout0).wait()


def _sc_copy(buffer):
    mesh = plsc.VectorSubcoreMesh(core_axis_name="c", subcore_axis_name="s")
    fn = functools.partial(
        pl.kernel,
        mesh=mesh,
        out_type=jax.ShapeDtypeStruct((_N, _R, _C), jnp.float32),
        scratch_types=[
            pltpu.VMEM((1, _R, _C), jnp.float32),
            pltpu.VMEM((1, _R, _C), jnp.float32),
            pltpu.SemaphoreType.DMA,
            pltpu.SemaphoreType.DMA,
            pltpu.SemaphoreType.DMA,
            pltpu.SemaphoreType.DMA,
        ],
    )(_sc_copy_body)
    return fn(buffer)


def _tc_body(idx_s, w_s, u_s, buf, noise, ns_row, idx_col, idx_row, u_col,
             out, new_buf, steps_out, new_ns_out, row_v, sem_in, sem_out,
             sem_big):
    big = pltpu.make_async_copy(buf, new_buf, sem_big)
    big.start()
    def gather_group(gg, carry):
        base = gg * _Q
        for q in range(_Q):
            b = base + q
            reinit = u_s[b] < _REINIT_P

            @pl.when(reinit)
            def _():
                pltpu.make_async_copy(noise.at[pl.ds(b, 1)],
                                      row_v.at[q], sem_in.at[q]).start()

            @pl.when(jnp.logical_not(reinit))
            def _():
                pltpu.make_async_copy(buf.at[pl.ds(idx_s[b], 1)],
                                      row_v.at[q], sem_in.at[q]).start()

        for q in range(_Q):
            pltpu.make_async_copy(noise.at[pl.ds(0, 1)],
                                  row_v.at[q], sem_in.at[q]).wait()
        for q in range(_Q):
            pltpu.make_async_copy(row_v.at[q],
                                  out.at[pl.ds(base + q, 1)],
                                  sem_out.at[q]).start()
        for q in range(_Q):
            pltpu.make_async_copy(row_v.at[q],
                                  out.at[pl.ds(base + q, 1)],
                                  sem_out.at[q]).wait()
        return carry

    jax.lax.fori_loop(0, _B // _Q, gather_group, 0)

    ns = ns_row[...]
    ic = idx_col[...]
    ir = idx_row[...]
    rc = u_col[...] < _REINIT_P
    col_ids = jax.lax.broadcasted_iota(jnp.int32, (_B, _N), 1)
    onehot = ic == col_ids
    g = jnp.sum(jnp.where(onehot, ns, 0.0), axis=1, keepdims=True)
    steps = jnp.where(rc, 0.0, g)
    steps_out[...] = steps
    bi = jax.lax.broadcasted_iota(jnp.int32, (_B, _B), 0)
    bj = jax.lax.broadcasted_iota(jnp.int32, (_B, _B), 1)
    later_same = (ic == ir) & (bj > bi)
    winner = jnp.logical_not(jnp.any(later_same, axis=1, keepdims=True))
    sc_mask = onehot & winner
    contrib = jnp.sum(jnp.where(sc_mask, steps, 0.0), axis=0, keepdims=True)
    written = jnp.any(sc_mask, axis=0, keepdims=True)
    new_ns_out[...] = jnp.where(written, contrib, ns)

    big.wait()

    def scatter_one(b, carry):
        cond = (u_s[b] < _REINIT_P) & (w_s[b] == b)

        @pl.when(cond)
        def _():
            pltpu.make_async_copy(noise.at[pl.ds(b, 1)],
                                  row_v.at[0], sem_in.at[0]).start()
            pltpu.make_async_copy(noise.at[pl.ds(b, 1)],
                                  row_v.at[0], sem_in.at[0]).wait()
            pltpu.make_async_copy(row_v.at[0],
                                  new_buf.at[pl.ds(idx_s[b], 1)],
                                  sem_out.at[0]).start()
            pltpu.make_async_copy(row_v.at[0],
                                  new_buf.at[pl.ds(idx_s[b], 1)],
                                  sem_out.at[0]).wait()

        return carry

    jax.lax.fori_loop(0, _B, scatter_one, 0)


def kernel(buffer, buffer_numsteps, noise, u, idx):
    idx = idx.astype(jnp.int32)
    eq = idx[:, None] == idx[None, :]
    w = jnp.max(jnp.where(eq, jnp.arange(_B, dtype=jnp.int32)[None, :], -1), axis=1)

    smem = pltpu.MemorySpace.SMEM
    hbm = pltpu.MemorySpace.HBM
    out, new_buffer, steps, new_ns = pl.pallas_call(
        _tc_body,
        in_specs=[
            pl.BlockSpec(memory_space=smem),
            pl.BlockSpec(memory_space=smem),
            pl.BlockSpec(memory_space=smem),
            pl.BlockSpec(memory_space=hbm),
            pl.BlockSpec(memory_space=hbm),
            pl.BlockSpec((1, _N), lambda: (0, 0)),
            pl.BlockSpec((_B, 1), lambda: (0, 0)),
            pl.BlockSpec((1, _B), lambda: (0, 0)),
            pl.BlockSpec((_B, 1), lambda: (0, 0)),
        ],
        out_specs=[
            pl.BlockSpec(memory_space=hbm),
            pl.BlockSpec(memory_space=hbm),
            pl.BlockSpec((_B, 1), lambda: (0, 0)),
            pl.BlockSpec((1, _N), lambda: (0, 0)),
        ],
        out_shape=[
            jax.ShapeDtypeStruct((_B, _R, _C), jnp.float32),
            jax.ShapeDtypeStruct((_N, _R, _C), jnp.float32),
            jax.ShapeDtypeStruct((_B, 1), jnp.float32),
            jax.ShapeDtypeStruct((1, _N), jnp.float32),
        ],
        scratch_shapes=[
            pltpu.VMEM((_Q, 1, _R, _C), jnp.float32),
            pltpu.SemaphoreType.DMA((_Q,)),
            pltpu.SemaphoreType.DMA((_Q,)),
            pltpu.SemaphoreType.DMA,
        ],
    )(idx, w, u, buffer, noise, buffer_numsteps.reshape(1, _N),
      idx.reshape(_B, 1), idx.reshape(1, _B), u.reshape(_B, 1))

    return out, steps.reshape(_B), new_buffer, new_ns.reshape(_N)


# speedup vs baseline: 18.1664x; 18.1664x over previous
import functools

import jax
import jax.numpy as jnp
from jax import lax
from jax.experimental import pallas as pl
from jax.experimental.pallas import tpu as pltpu
from jax.experimental.pallas import tpu_sc as plsc

_REINIT_P = 0.05
_N, _R, _C = 10000, 250, 100
_B = 128
_Q = 8
_NW = 32
_J = _N // _NW
_TAIL = _N - _J * _NW


def _sc_copy_body(buf, dst, s0, s1, din0, din1, dout0, dout1):
    wid = lax.axis_index("s") * 2 + lax.axis_index("c")

    def pair(p, carry):
        r0 = (2 * p) * _NW + wid
        r1 = (2 * p + 1) * _NW + wid
        c0 = pltpu.async_copy(buf.at[pl.ds(r0, 1)], s0, din0)
        c1 = pltpu.async_copy(buf.at[pl.ds(r1, 1)], s1, din1)
        c0.wait()
        o0 = pltpu.async_copy(s0, dst.at[pl.ds(r0, 1)], dout0)
        c1.wait()
        o1 = pltpu.async_copy(s1, dst.at[pl.ds(r1, 1)], dout1)
        o0.wait()
        o1.wait()
        return carry

    jax.lax.fori_loop(0, _J // 2, pair, 0)

    @pl.when(wid < _TAIL)
    def _():
        r = _J * _NW + wid
        pltpu.async_copy(buf.at[pl.ds(r, 1)], s0, din0).wait()
        pltpu.async_copy(s0, dst.at[pl.ds(r, 1)], dout0).wait()


def _sc_copy(buffer):
    mesh = plsc.VectorSubcoreMesh(core_axis_name="c", subcore_axis_name="s")
    fn = functools.partial(
        pl.kernel,
        mesh=mesh,
        out_type=jax.ShapeDtypeStruct((_N, _R, _C), jnp.float32),
        scratch_types=[
            pltpu.VMEM((1, _R, _C), jnp.float32),
            pltpu.VMEM((1, _R, _C), jnp.float32),
            pltpu.SemaphoreType.DMA,
            pltpu.SemaphoreType.DMA,
            pltpu.SemaphoreType.DMA,
            pltpu.SemaphoreType.DMA,
        ],
    )(_sc_copy_body)
    return fn(buffer)


def _tc_body(idx_s, w_s, u_s, buf, noise, ns_row, idx_col, idx_row, u_col,
             out, new_buf, steps_out, new_ns_out, row_v, sem_in, sem_out,
             sem_big):
    del sem_big
    def gather_group(gg, carry):
        base = gg * _Q
        for q in range(_Q):
            b = base + q
            reinit = u_s[b] < _REINIT_P

            @pl.when(reinit)
            def _():
                pltpu.make_async_copy(noise.at[pl.ds(b, 1)],
                                      row_v.at[q], sem_in.at[q]).start()

            @pl.when(jnp.logical_not(reinit))
            def _():
                pltpu.make_async_copy(buf.at[pl.ds(idx_s[b], 1)],
                                      row_v.at[q], sem_in.at[q]).start()

        for q in range(_Q):
            pltpu.make_async_copy(noise.at[pl.ds(0, 1)],
                                  row_v.at[q], sem_in.at[q]).wait()
        for q in range(_Q):
            pltpu.make_async_copy(row_v.at[q],
                                  out.at[pl.ds(base + q, 1)],
                                  sem_out.at[q]).start()
        for q in range(_Q):
            pltpu.make_async_copy(row_v.at[q],
                                  out.at[pl.ds(base + q, 1)],
                                  sem_out.at[q]).wait()
        return carry

    jax.lax.fori_loop(0, _B // _Q, gather_group, 0)

    ns = ns_row[...]
    ic = idx_col[...]
    ir = idx_row[...]
    rc = u_col[...] < _REINIT_P
    col_ids = jax.lax.broadcasted_iota(jnp.int32, (_B, _N), 1)
    onehot = ic == col_ids
    g = jnp.sum(jnp.where(onehot, ns, 0.0), axis=1, keepdims=True)
    steps = jnp.where(rc, 0.0, g)
    steps_out[...] = steps
    bi = jax.lax.broadcasted_iota(jnp.int32, (_B, _B), 0)
    bj = jax.lax.broadcasted_iota(jnp.int32, (_B, _B), 1)
    later_same = (ic == ir) & (bj > bi)
    winner = jnp.logical_not(jnp.any(later_same, axis=1, keepdims=True))
    sc_mask = onehot & winner
    contrib = jnp.sum(jnp.where(sc_mask, steps, 0.0), axis=0, keepdims=True)
    written = jnp.any(sc_mask, axis=0, keepdims=True)
    new_ns_out[...] = jnp.where(written, contrib, ns)

    def scatter_one(b, carry):
        cond = (u_s[b] < _REINIT_P) & (w_s[b] == b)

        @pl.when(cond)
        def _():
            pltpu.make_async_copy(noise.at[pl.ds(b, 1)],
                                  row_v.at[0], sem_in.at[0]).start()
            pltpu.make_async_copy(noise.at[pl.ds(b, 1)],
                                  row_v.at[0], sem_in.at[0]).wait()
            pltpu.make_async_copy(row_v.at[0],
                                  new_buf.at[pl.ds(idx_s[b], 1)],
                                  sem_out.at[0]).start()
            pltpu.make_async_copy(row_v.at[0],
                                  new_buf.at[pl.ds(idx_s[b], 1)],
                                  sem_out.at[0]).wait()

        return carry

    jax.lax.fori_loop(0, _B, scatter_one, 0)


def kernel(buffer, buffer_numsteps, noise, u, idx):
    idx = idx.astype(jnp.int32)
    eq = idx[:, None] == idx[None, :]
    w = jnp.max(jnp.where(eq, jnp.arange(_B, dtype=jnp.int32)[None, :], -1), axis=1)

    smem = pltpu.MemorySpace.SMEM
    hbm = pltpu.MemorySpace.HBM
    out, new_buffer, steps, new_ns = pl.pallas_call(
        _tc_body,
        in_specs=[
            pl.BlockSpec(memory_space=smem),
            pl.BlockSpec(memory_space=smem),
            pl.BlockSpec(memory_space=smem),
            pl.BlockSpec(memory_space=hbm),
            pl.BlockSpec(memory_space=hbm),
            pl.BlockSpec((1, _N), lambda: (0, 0)),
            pl.BlockSpec((_B, 1), lambda: (0, 0)),
            pl.BlockSpec((1, _B), lambda: (0, 0)),
            pl.BlockSpec((_B, 1), lambda: (0, 0)),
        ],
        out_specs=[
            pl.BlockSpec(memory_space=hbm),
            pl.BlockSpec(memory_space=hbm),
            pl.BlockSpec((_B, 1), lambda: (0, 0)),
            pl.BlockSpec((1, _N), lambda: (0, 0)),
        ],
        out_shape=[
            jax.ShapeDtypeStruct((_B, _R, _C), jnp.float32),
            jax.ShapeDtypeStruct((_N, _R, _C), jnp.float32),
            jax.ShapeDtypeStruct((_B, 1), jnp.float32),
            jax.ShapeDtypeStruct((1, _N), jnp.float32),
        ],
        input_output_aliases={3: 1},
        scratch_shapes=[
            pltpu.VMEM((_Q, 1, _R, _C), jnp.float32),
            pltpu.SemaphoreType.DMA((_Q,)),
            pltpu.SemaphoreType.DMA((_Q,)),
            pltpu.SemaphoreType.DMA,
        ],
    )(idx, w, u, buffer, noise, buffer_numsteps.reshape(1, _N),
      idx.reshape(_B, 1), idx.reshape(1, _B), u.reshape(_B, 1))

    return out, steps.reshape(_B), new_buffer, new_ns.reshape(_N)
